# baseline (speedup 1.0000x reference)
import functools

import jax
import jax.numpy as jnp
from jax import lax
from jax.experimental import pallas as pl
from jax.experimental.pallas import tpu as pltpu
from jax.experimental.pallas import tpu_sc as plsc

N_NODES = 10000
N_EDGES = 320000
D = 128

NC = 2
NS = 16
NW = NC * NS
L = 16

NACC = 10240
RPT = NACC // NS
NPW = NACC // NW
CHUNK = 128
K = 1
EPT = 10240
NCHUNK = EPT // CHUNK
NGRP = NCHUNK // K
E_PAD = EPT * NW

_MESH = dict(core_axis_name="c", subcore_axis_name="s", num_cores=NC,
             num_subcores=NS)


def _worker_ids():
    cid = lax.axis_index("c")
    sid = lax.axis_index("s")
    return cid, sid, cid * NS + sid


HB = 8


def _hist_body(col_hbm, out_hbm, cidx, ones, zrow, hist_sh, sem):
    cid, sid, w = _worker_ids()
    for i in range(CHUNK // L):
        ones[pl.ds(i * L, L)] = jnp.ones((L,), jnp.float32)
    for i in range(RPT // L):
        zrow[pl.ds(i * L, L)] = jnp.zeros((L,), jnp.float32)
    pltpu.sync_copy(zrow, hist_sh.at[pl.ds(sid * RPT, RPT)])
    plsc.subcore_barrier()

    crow = w * NCHUNK

    def step(g, carry):
        pltpu.sync_copy(col_hbm.at[pl.ds(crow + g * HB, HB)], cidx)
        ds = [pltpu.async_copy(ones, hist_sh.at[cidx.at[b]], sem, add=True)
              for b in range(HB)]
        for d in ds:
            d.wait()
        return carry

    lax.fori_loop(0, NCHUNK // HB, step, 0)
    plsc.subcore_barrier()
    pltpu.sync_copy(hist_sh.at[pl.ds(sid * RPT, RPT)],
                    out_hbm.at[pl.ds(cid * NACC + sid * RPT, RPT)])


_hist_call = functools.partial(
    pl.kernel,
    out_type=jax.ShapeDtypeStruct((NC * NACC,), jnp.float32),
    mesh=plsc.VectorSubcoreMesh(**_MESH),
    scratch_types=[
        pltpu.VMEM((HB, CHUNK), jnp.int32),
        pltpu.VMEM((CHUNK,), jnp.float32),
        pltpu.VMEM((RPT,), jnp.float32),
        pltpu.VMEM_SHARED((NACC,), jnp.float32),
        pltpu.SemaphoreType.DMA,
    ],
)(_hist_body)


def _dinv_body(hist_hbm, out_hbm, h0, h1, dv):
    _, _, w = _worker_ids()
    base = w * NPW
    pltpu.sync_copy(hist_hbm.at[pl.ds(base, NPW)], h0)
    pltpu.sync_copy(hist_hbm.at[pl.ds(NACC + base, NPW)], h1)

    def step(i, carry):
        sl = pl.ds(i * L, L)
        d = h0[sl] + h1[sl] + 2.0
        bits = lax.bitcast_convert_type(d, jnp.int32)
        y = lax.bitcast_convert_type(
            jnp.int32(0x5F3759DF) - (bits >> 1), jnp.float32)
        for _ in range(3):
            y = y * (1.5 - 0.5 * d * y * y)
        dv[sl] = y
        return carry

    lax.fori_loop(0, NPW // L, step, 0)
    pltpu.sync_copy(dv, out_hbm.at[pl.ds(base, NPW)])


_dinv_call = functools.partial(
    pl.kernel,
    out_type=jax.ShapeDtypeStruct((NACC,), jnp.float32),
    mesh=plsc.VectorSubcoreMesh(**_MESH),
    scratch_types=[
        pltpu.VMEM((NPW,), jnp.float32),
        pltpu.VMEM((NPW,), jnp.float32),
        pltpu.VMEM((NPW,), jnp.float32),
    ],
)(_dinv_body)


def _edge_body(g_hbm, row_hbm, col_hbm, out_hbm, ridx, cidx, cidxs, rows, zv,
               acc_sh, semi0, semi1, semg, sems0, sems1):
    cid, sid, w = _worker_ids()
    semi = (semi0, semi1)
    sems = (sems0, sems1)
    for i in range(16):
        for j in range(D // L):
            zv[i, pl.ds(j * L, L)] = jnp.zeros((L,), jnp.float32)

    def zstep(k, carry):
        r = pl.multiple_of(sid * RPT + k * 16, 8)
        pltpu.sync_copy(zv, acc_sh.at[pl.ds(r, 16)])
        return carry

    lax.fori_loop(0, RPT // 16, zstep, 0)
    plsc.subcore_barrier()

    crow = w * NCHUNK

    pltpu.async_copy(row_hbm.at[pl.ds(crow, K)], ridx.at[0], semi0)
    pltpu.async_copy(col_hbm.at[pl.ds(crow, K)], cidx.at[0], semi0)

    def step(g2, carry):
        for p in range(2):
            g = 2 * g2 + p
            rp, cp, csp = ridx.at[p], cidx.at[p], cidxs.at[p]
            rwp = rows.at[p]

            @pl.when(g >= 2)
            def _drain():
                for b in range(K):
                    pltpu.make_async_copy(
                        rwp.at[b], acc_sh.at[csp.at[b]], sems[p]).wait()

            pltpu.make_async_copy(row_hbm.at[pl.ds(crow, K)], rp,
                                  semi[p]).wait()
            pltpu.make_async_copy(col_hbm.at[pl.ds(crow, K)], cp,
                                  semi[p]).wait()

            @pl.when(g + 1 < NGRP)
            def _prefetch():
                off = crow + (g + 1) * K
                pltpu.async_copy(row_hbm.at[pl.ds(off, K)], ridx.at[1 - p],
                                 semi[1 - p])
                pltpu.async_copy(col_hbm.at[pl.ds(off, K)], cidx.at[1 - p],
                                 semi[1 - p])

            gd = [pltpu.async_copy(g_hbm.at[rp.at[b]], rwp.at[b], semg)
                  for b in range(K)]
            for b in range(K):
                gd[b].wait()
                for j in range(CHUNK // L):
                    csp[b, pl.ds(j * L, L)] = cp[b, pl.ds(j * L, L)]
                pltpu.async_copy(rwp.at[b], acc_sh.at[csp.at[b]], sems[p],
                                 add=True)
        return carry

    lax.fori_loop(0, NGRP // 2, step, 0)
    for p in range(2):
        for b in range(K):
            pltpu.make_async_copy(rows.at[p].at[b],
                                  acc_sh.at[cidxs.at[p].at[b]],
                                  sems[p]).wait()
    plsc.subcore_barrier()

    def wstep(k, carry):
        r = pl.multiple_of(sid * RPT + k * 64, 8)
        pltpu.sync_copy(acc_sh.at[pl.ds(r, 64)], out_hbm.at[cid, pl.ds(r, 64)])
        return carry

    lax.fori_loop(0, RPT // 64, wstep, 0)


_edge_call = functools.partial(
    pl.kernel,
    out_type=jax.ShapeDtypeStruct((NC, NACC, D), jnp.float32),
    mesh=plsc.VectorSubcoreMesh(**_MESH),
    scratch_types=[
        pltpu.VMEM((2, K, CHUNK), jnp.int32),
        pltpu.VMEM((2, K, CHUNK), jnp.int32),
        pltpu.VMEM((2, K, CHUNK), jnp.int32),
        pltpu.VMEM((2, K, CHUNK, D), jnp.float32),
        pltpu.VMEM((16, D), jnp.float32),
        pltpu.VMEM_SHARED((NACC, D), jnp.float32),
        pltpu.SemaphoreType.DMA,
        pltpu.SemaphoreType.DMA,
        pltpu.SemaphoreType.DMA,
        pltpu.SemaphoreType.DMA,
        pltpu.SemaphoreType.DMA,
    ],
)(_edge_body)


BM = 1000
_GRID = (N_NODES // BM,)


def _mmp_body(x_ref, w_ref, t_ref):
    t_ref[...] = jnp.dot(x_ref[...], w_ref[...],
                         preferred_element_type=jnp.float32)


def _scale_body(dinv_ref, t_ref, g_ref):
    g_ref[...] = dinv_ref[...] * t_ref[...]


def _mm2_body(dinv_ref, acc_ref, g1_ref, b1_ref, w_ref, g2_ref):
    s = acc_ref[0] + acc_ref[1] + 2.0 * g1_ref[...]
    h1 = jnp.maximum(dinv_ref[...] * s + b1_ref[...], 0.0)
    g2_ref[...] = dinv_ref[...] * jnp.dot(
        h1, w_ref[...], preferred_element_type=jnp.float32)


def _fin_body(dinv_ref, acc_ref, g2_ref, b2_ref, o_ref):
    s = acc_ref[0] + acc_ref[1] + 2.0 * g2_ref[...]
    o_ref[...] = dinv_ref[...] * s + b2_ref[...]


_col_spec = pl.BlockSpec((BM, 1), lambda i: (i, 0))
_row_spec = pl.BlockSpec((BM, D), lambda i: (i, 0))
_acc_spec = pl.BlockSpec((NC, BM, D), lambda i: (0, i, 0))
_w_spec = pl.BlockSpec((D, D), lambda i: (0, 0))
_b_spec = pl.BlockSpec((1, D), lambda i: (0, 0))

_mmp = pl.pallas_call(
    _mmp_body, grid=_GRID,
    in_specs=[_row_spec, _w_spec],
    out_specs=_row_spec,
    out_shape=jax.ShapeDtypeStruct((N_NODES, D), jnp.float32))

_scale = pl.pallas_call(
    _scale_body, grid=_GRID,
    in_specs=[_col_spec, _row_spec],
    out_specs=_row_spec,
    out_shape=jax.ShapeDtypeStruct((N_NODES, D), jnp.float32))

_mm2 = pl.pallas_call(
    _mm2_body, grid=_GRID,
    in_specs=[_col_spec, _acc_spec, _row_spec, _b_spec, _w_spec],
    out_specs=_row_spec,
    out_shape=jax.ShapeDtypeStruct((N_NODES, D), jnp.float32))

_fin = pl.pallas_call(
    _fin_body, grid=_GRID,
    in_specs=[_col_spec, _acc_spec, _row_spec, _b_spec],
    out_specs=_row_spec,
    out_shape=jax.ShapeDtypeStruct((N_NODES, D), jnp.float32))


def kernel(x, edge_index, W1, b1, W2, b2):
    ei = edge_index.astype(jnp.int32)
    npad = E_PAD - N_EDGES
    fill = jnp.arange(npad, dtype=jnp.int32)
    row_pad = jnp.concatenate([ei[0], fill % N_NODES]).reshape(-1, CHUNK)
    col_pad = jnp.concatenate(
        [ei[1], N_NODES + fill % (NACC - N_NODES)]).reshape(-1, CHUNK)

    t1 = _mmp(x, W1)
    hist = _hist_call(col_pad)
    dinv = _dinv_call(hist)
    dinv_col = dinv[:N_NODES, None]

    g1 = _scale(dinv_col, t1)
    acc1 = _edge_call(g1, row_pad, col_pad)
    g2 = _mm2(dinv_col, acc1, g1, b1.reshape(1, D), W2)
    acc2 = _edge_call(g2, row_pad, col_pad)
    return _fin(dinv_col, acc2, g2, b2.reshape(1, D))


# speedup vs baseline: 1.0210x; 1.0210x over previous
import functools

import jax
import jax.numpy as jnp
from jax import lax
from jax.experimental import pallas as pl
from jax.experimental.pallas import tpu as pltpu
from jax.experimental.pallas import tpu_sc as plsc

N_NODES = 10000
N_EDGES = 320000
D = 128

NC = 2
NS = 16
NW = NC * NS
L = 16

NACC = 10240
RPT = NACC // NS
CHUNK = 128
EPT = 10240
NGRP = EPT // CHUNK
E_PAD = EPT * NW

_MESH = dict(core_axis_name="c", subcore_axis_name="s", num_cores=NC,
             num_subcores=NS)


def _worker_ids():
    cid = lax.axis_index("c")
    sid = lax.axis_index("s")
    return cid, sid, cid * NS + sid


HB = 8


def _hist_body(col_hbm, out_hbm, cidx, ones, zrow, hist_sh, sem):
    cid, sid, w = _worker_ids()
    for i in range(CHUNK // L):
        ones[pl.ds(i * L, L)] = jnp.ones((L,), jnp.float32)
    for i in range(RPT // L):
        zrow[pl.ds(i * L, L)] = jnp.zeros((L,), jnp.float32)
    pltpu.sync_copy(zrow, hist_sh.at[pl.ds(sid * RPT, RPT)])
    plsc.subcore_barrier()

    crow = w * NGRP

    def step(g, carry):
        pltpu.sync_copy(col_hbm.at[pl.ds(crow + g * HB, HB)], cidx)
        ds = [pltpu.async_copy(ones, hist_sh.at[cidx.at[b]], sem, add=True)
              for b in range(HB)]
        for d in ds:
            d.wait()
        return carry

    lax.fori_loop(0, NGRP // HB, step, 0)
    plsc.subcore_barrier()
    pltpu.sync_copy(hist_sh.at[pl.ds(sid * RPT, RPT)],
                    out_hbm.at[pl.ds(cid * NACC + sid * RPT, RPT)])


_hist_call = functools.partial(
    pl.kernel,
    out_type=jax.ShapeDtypeStruct((NC * NACC,), jnp.float32),
    mesh=plsc.VectorSubcoreMesh(**_MESH),
    scratch_types=[
        pltpu.VMEM((HB, CHUNK), jnp.int32),
        pltpu.VMEM((CHUNK,), jnp.float32),
        pltpu.VMEM((RPT,), jnp.float32),
        pltpu.VMEM_SHARED((NACC,), jnp.float32),
        pltpu.SemaphoreType.DMA,
    ],
)(_hist_body)


def _edge_body(g_hbm, rc_hbm, out_hbm, idx, cidxs, rows, zv, acc_sh,
               semi0, semi1, semg, sems0, sems1):
    cid, sid, w = _worker_ids()
    semi = (semi0, semi1)
    sems = (sems0, sems1)
    for i in range(16):
        for j in range(D // L):
            zv[i, pl.ds(j * L, L)] = jnp.zeros((L,), jnp.float32)

    def zstep(k, carry):
        r = pl.multiple_of(sid * RPT + k * 16, 8)
        pltpu.sync_copy(zv, acc_sh.at[pl.ds(r, 16)])
        return carry

    lax.fori_loop(0, RPT // 16, zstep, 0)
    plsc.subcore_barrier()

    crow = w * NGRP

    pltpu.async_copy(rc_hbm.at[crow], idx.at[0], semi0)

    def step(g2, carry):
        for p in range(2):
            g = 2 * g2 + p
            ip, csp, rwp = idx.at[p], cidxs.at[p], rows.at[p]

            @pl.when(g >= 2)
            def _drain():
                pltpu.make_async_copy(rwp, acc_sh.at[csp], sems[p]).wait()

            pltpu.make_async_copy(rc_hbm.at[crow], ip, semi[p]).wait()

            @pl.when(g + 1 < NGRP)
            def _prefetch():
                pltpu.async_copy(rc_hbm.at[crow + g + 1], idx.at[1 - p],
                                 semi[1 - p])

            gd = pltpu.async_copy(g_hbm.at[ip.at[0]], rwp, semg)
            gd.wait()
            for j in range(CHUNK // L):
                csp[pl.ds(j * L, L)] = ip[1, pl.ds(j * L, L)]
            pltpu.async_copy(rwp, acc_sh.at[csp], sems[p], add=True)
        return carry

    lax.fori_loop(0, NGRP // 2, step, 0)
    for p in range(2):
        pltpu.make_async_copy(rows.at[p], acc_sh.at[cidxs.at[p]],
                              sems[p]).wait()
    plsc.subcore_barrier()

    def wstep(k, carry):
        r = pl.multiple_of(sid * RPT + k * 64, 8)
        pltpu.sync_copy(acc_sh.at[pl.ds(r, 64)], out_hbm.at[cid, pl.ds(r, 64)])
        return carry

    lax.fori_loop(0, RPT // 64, wstep, 0)


_edge_call = functools.partial(
    pl.kernel,
    out_type=jax.ShapeDtypeStruct((NC, NACC, D), jnp.float32),
    mesh=plsc.VectorSubcoreMesh(**_MESH),
    scratch_types=[
        pltpu.VMEM((2, 2, CHUNK), jnp.int32),
        pltpu.VMEM((2, CHUNK), jnp.int32),
        pltpu.VMEM((2, CHUNK, D), jnp.float32),
        pltpu.VMEM((16, D), jnp.float32),
        pltpu.VMEM_SHARED((NACC, D), jnp.float32),
        pltpu.SemaphoreType.DMA,
        pltpu.SemaphoreType.DMA,
        pltpu.SemaphoreType.DMA,
        pltpu.SemaphoreType.DMA,
        pltpu.SemaphoreType.DMA,
    ],
)(_edge_body)


BM = 1000
_GRID = (N_NODES // BM,)


def _dinv(h0_ref, h1_ref):
    return lax.rsqrt(h0_ref[...] + h1_ref[...] + 2.0)


def _mm1_body(h0_ref, h1_ref, x_ref, w_ref, g_ref):
    g_ref[...] = _dinv(h0_ref, h1_ref) * jnp.dot(
        x_ref[...], w_ref[...], preferred_element_type=jnp.float32)


def _mm2_body(h0_ref, h1_ref, acc_ref, g1_ref, b1_ref, w_ref, g2_ref):
    dinv = _dinv(h0_ref, h1_ref)
    s = acc_ref[0] + acc_ref[1] + 2.0 * g1_ref[...]
    h1 = jnp.maximum(dinv * s + b1_ref[...], 0.0)
    g2_ref[...] = dinv * jnp.dot(
        h1, w_ref[...], preferred_element_type=jnp.float32)


def _fin_body(h0_ref, h1_ref, acc_ref, g2_ref, b2_ref, o_ref):
    s = acc_ref[0] + acc_ref[1] + 2.0 * g2_ref[...]
    o_ref[...] = _dinv(h0_ref, h1_ref) * s + b2_ref[...]


_col_spec = pl.BlockSpec((BM, 1), lambda i: (i, 0))
_row_spec = pl.BlockSpec((BM, D), lambda i: (i, 0))
_acc_spec = pl.BlockSpec((NC, BM, D), lambda i: (0, i, 0))
_w_spec = pl.BlockSpec((D, D), lambda i: (0, 0))
_b_spec = pl.BlockSpec((1, D), lambda i: (0, 0))

_mm1 = pl.pallas_call(
    _mm1_body, grid=_GRID,
    in_specs=[_col_spec, _col_spec, _row_spec, _w_spec],
    out_specs=_row_spec,
    out_shape=jax.ShapeDtypeStruct((N_NODES, D), jnp.float32))

_mm2 = pl.pallas_call(
    _mm2_body, grid=_GRID,
    in_specs=[_col_spec, _col_spec, _acc_spec, _row_spec, _b_spec, _w_spec],
    out_specs=_row_spec,
    out_shape=jax.ShapeDtypeStruct((N_NODES, D), jnp.float32))

_fin = pl.pallas_call(
    _fin_body, grid=_GRID,
    in_specs=[_col_spec, _col_spec, _acc_spec, _row_spec, _b_spec],
    out_specs=_row_spec,
    out_shape=jax.ShapeDtypeStruct((N_NODES, D), jnp.float32))


def kernel(x, edge_index, W1, b1, W2, b2):
    ei = edge_index.astype(jnp.int32)
    npad = E_PAD - N_EDGES
    fill = jnp.arange(npad, dtype=jnp.int32)
    row2d = jnp.concatenate([ei[0], fill % N_NODES]).reshape(-1, CHUNK)
    col2d = jnp.concatenate(
        [ei[1], N_NODES + fill % (NACC - N_NODES)]).reshape(-1, CHUNK)
    rc = jnp.stack([row2d, col2d], axis=1)

    hist = _hist_call(col2d).reshape(NC, NACC)
    h0 = hist[0, :N_NODES, None]
    h1 = hist[1, :N_NODES, None]

    g1 = _mm1(h0, h1, x, W1)
    acc1 = _edge_call(g1, rc)
    g2 = _mm2(h0, h1, acc1, g1, b1.reshape(1, D), W2)
    acc2 = _edge_call(g2, rc)
    return _fin(h0, h1, acc2, g2, b2.reshape(1, D))


# speedup vs baseline: 1.0364x; 1.0151x over previous
import functools

import jax
import jax.numpy as jnp
from jax import lax
from jax.experimental import pallas as pl
from jax.experimental.pallas import tpu as pltpu
from jax.experimental.pallas import tpu_sc as plsc

N_NODES = 10000
N_EDGES = 320000
D = 128

NC = 2
NS = 16
NW = NC * NS
L = 16

NACC = 10240
RPT = NACC // NS
CHUNK = 128
EPT = 10240
NGRP = EPT // CHUNK
E_PAD = EPT * NW

_MESH = dict(core_axis_name="c", subcore_axis_name="s", num_cores=NC,
             num_subcores=NS)


def _worker_ids():
    cid = lax.axis_index("c")
    sid = lax.axis_index("s")
    return cid, sid, cid * NS + sid


HB = 8


def _hist_body(col_hbm, out_hbm, cidx, ones, zrow, hist_sh, sem):
    cid, sid, w = _worker_ids()
    for i in range(CHUNK // L):
        ones[pl.ds(i * L, L)] = jnp.ones((L,), jnp.float32)
    for i in range(RPT // L):
        zrow[pl.ds(i * L, L)] = jnp.zeros((L,), jnp.float32)
    pltpu.sync_copy(zrow, hist_sh.at[pl.ds(sid * RPT, RPT)])
    plsc.subcore_barrier()

    crow = w * NGRP

    def step(g, carry):
        pltpu.sync_copy(col_hbm.at[pl.ds(crow + g * HB, HB)], cidx)
        ds = [pltpu.async_copy(ones, hist_sh.at[cidx.at[b]], sem, add=True)
              for b in range(HB)]
        for d in ds:
            d.wait()
        return carry

    lax.fori_loop(0, NGRP // HB, step, 0)
    plsc.subcore_barrier()
    pltpu.sync_copy(hist_sh.at[pl.ds(sid * RPT, RPT)],
                    out_hbm.at[pl.ds(cid * NACC + sid * RPT, RPT)])


_hist_call = functools.partial(
    pl.kernel,
    out_type=jax.ShapeDtypeStruct((NC * NACC,), jnp.float32),
    mesh=plsc.VectorSubcoreMesh(**_MESH),
    scratch_types=[
        pltpu.VMEM((HB, CHUNK), jnp.int32),
        pltpu.VMEM((CHUNK,), jnp.float32),
        pltpu.VMEM((RPT,), jnp.float32),
        pltpu.VMEM_SHARED((NACC,), jnp.float32),
        pltpu.SemaphoreType.DMA,
    ],
)(_hist_body)


def _edge_body(g_hbm, rc_hbm, out_hbm, idx, cidxs, rows, zv, acc_sh,
               semi0, semi1, semg, sems0, sems1):
    cid, sid, w = _worker_ids()
    semi = (semi0, semi1)
    sems = (sems0, sems1)
    for i in range(16):
        for j in range(D // L):
            zv[i, pl.ds(j * L, L)] = jnp.zeros((L,), jnp.float32)

    zd = [pltpu.async_copy(zv, acc_sh.at[pl.ds(sid * RPT + k * 16, 16)],
                           semg)
          for k in range(RPT // 16)]
    for d in zd:
        d.wait()
    plsc.subcore_barrier()

    crow = w * NGRP

    pltpu.async_copy(rc_hbm.at[crow], idx.at[0], semi0)

    def step(g2, carry):
        for p in range(2):
            g = 2 * g2 + p
            ip, csp, rwp = idx.at[p], cidxs.at[p], rows.at[p]

            @pl.when(g >= 2)
            def _drain():
                pltpu.make_async_copy(rwp, acc_sh.at[csp], sems[p]).wait()

            pltpu.make_async_copy(rc_hbm.at[crow], ip, semi[p]).wait()

            @pl.when(g + 1 < NGRP)
            def _prefetch():
                pltpu.async_copy(rc_hbm.at[crow + g + 1], idx.at[1 - p],
                                 semi[1 - p])

            gd = pltpu.async_copy(g_hbm.at[ip.at[0]], rwp, semg)
            gd.wait()
            for j in range(CHUNK // L):
                csp[pl.ds(j * L, L)] = ip[1, pl.ds(j * L, L)]
            pltpu.async_copy(rwp, acc_sh.at[csp], sems[p], add=True)
        return carry

    lax.fori_loop(0, NGRP // 2, step, 0)
    for p in range(2):
        pltpu.make_async_copy(rows.at[p], acc_sh.at[cidxs.at[p]],
                              sems[p]).wait()
    plsc.subcore_barrier()
    pltpu.sync_copy(acc_sh.at[pl.ds(sid * RPT, RPT)],
                    out_hbm.at[cid, pl.ds(sid * RPT, RPT)])


_edge_call = functools.partial(
    pl.kernel,
    out_type=jax.ShapeDtypeStruct((NC, NACC, D), jnp.float32),
    mesh=plsc.VectorSubcoreMesh(**_MESH),
    scratch_types=[
        pltpu.VMEM((2, 2, CHUNK), jnp.int32),
        pltpu.VMEM((2, CHUNK), jnp.int32),
        pltpu.VMEM((2, CHUNK, D), jnp.float32),
        pltpu.VMEM((16, D), jnp.float32),
        pltpu.VMEM_SHARED((NACC, D), jnp.float32),
        pltpu.SemaphoreType.DMA,
        pltpu.SemaphoreType.DMA,
        pltpu.SemaphoreType.DMA,
        pltpu.SemaphoreType.DMA,
        pltpu.SemaphoreType.DMA,
    ],
)(_edge_body)


BM = 1000
_GRID = (N_NODES // BM,)


def _dinv(h0_ref, h1_ref):
    return lax.rsqrt(h0_ref[...] + h1_ref[...] + 2.0)


def _mm1_body(h0_ref, h1_ref, x_ref, w_ref, g_ref):
    g_ref[...] = _dinv(h0_ref, h1_ref) * jnp.dot(
        x_ref[...], w_ref[...], preferred_element_type=jnp.float32)


def _mm2_body(h0_ref, h1_ref, acc_ref, g1_ref, b1_ref, w_ref, g2_ref):
    dinv = _dinv(h0_ref, h1_ref)
    s = acc_ref[0] + acc_ref[1] + 2.0 * g1_ref[...]
    h1 = jnp.maximum(dinv * s + b1_ref[...], 0.0)
    g2_ref[...] = dinv * jnp.dot(
        h1, w_ref[...], preferred_element_type=jnp.float32)


def _fin_body(h0_ref, h1_ref, acc_ref, g2_ref, b2_ref, o_ref):
    s = acc_ref[0] + acc_ref[1] + 2.0 * g2_ref[...]
    o_ref[...] = _dinv(h0_ref, h1_ref) * s + b2_ref[...]


_col_spec = pl.BlockSpec((BM, 1), lambda i: (i, 0))
_row_spec = pl.BlockSpec((BM, D), lambda i: (i, 0))
_acc_spec = pl.BlockSpec((NC, BM, D), lambda i: (0, i, 0))
_w_spec = pl.BlockSpec((D, D), lambda i: (0, 0))
_b_spec = pl.BlockSpec((1, D), lambda i: (0, 0))

_mm1 = pl.pallas_call(
    _mm1_body, grid=_GRID,
    in_specs=[_col_spec, _col_spec, _row_spec, _w_spec],
    out_specs=_row_spec,
    out_shape=jax.ShapeDtypeStruct((N_NODES, D), jnp.float32))

_mm2 = pl.pallas_call(
    _mm2_body, grid=_GRID,
    in_specs=[_col_spec, _col_spec, _acc_spec, _row_spec, _b_spec, _w_spec],
    out_specs=_row_spec,
    out_shape=jax.ShapeDtypeStruct((N_NODES, D), jnp.float32))

_fin = pl.pallas_call(
    _fin_body, grid=_GRID,
    in_specs=[_col_spec, _col_spec, _acc_spec, _row_spec, _b_spec],
    out_specs=_row_spec,
    out_shape=jax.ShapeDtypeStruct((N_NODES, D), jnp.float32))


def kernel(x, edge_index, W1, b1, W2, b2):
    ei = edge_index.astype(jnp.int32)
    npad = E_PAD - N_EDGES
    fill = jnp.arange(npad, dtype=jnp.int32)
    row2d = jnp.concatenate([ei[0], fill % N_NODES]).reshape(-1, CHUNK)
    col2d = jnp.concatenate(
        [ei[1], N_NODES + fill % (NACC - N_NODES)]).reshape(-1, CHUNK)
    rc = jnp.stack([row2d, col2d], axis=1)

    hist = _hist_call(col2d).reshape(NC, NACC)
    h0 = hist[0, :N_NODES, None]
    h1 = hist[1, :N_NODES, None]

    g1 = _mm1(h0, h1, x, W1)
    acc1 = _edge_call(g1, rc)
    g2 = _mm2(h0, h1, acc1, g1, b1.reshape(1, D), W2)
    acc2 = _edge_call(g2, rc)
    return _fin(h0, h1, acc2, g2, b2.reshape(1, D))
